# baseline (speedup 1.0000x reference)
import dataclasses
import functools

import jax
import jax.numpy as jnp
import numpy as np
from jax import lax
from jax.experimental import pallas as pl
from jax.experimental.pallas import tpu as pltpu
from jax.experimental.pallas import tpu_sc as plsc

N = 10000
NP = 10240
E = 160000
ER = 1280
D_IN = 256
D_HID = 256
D_OUT = 128
RB = 1024
NB = NP // RB

_MESH = plsc.VectorSubcoreMesh(core_axis_name="c", subcore_axis_name="s")

_PAD_1D = N + np.arange(ER * 128 - E, dtype=np.int32) % (NP - N)
_PAD_EDGES = np.stack([_PAD_1D, _PAD_1D])

_NO_LAYOUT_CP = pltpu.CompilerParams()
if "needs_layout_passes" in pltpu.CompilerParams.__dataclass_fields__:
    _NO_LAYOUT_CP = dataclasses.replace(_NO_LAYOUT_CP,
                                        needs_layout_passes=False)


def _fill_f32(buf, nrows, ncols, val):
    v = jnp.full((16,), val, jnp.float32)

    @pl.loop(0, nrows)
    def _(r):
        @pl.loop(0, ncols, step=16)
        def _(c):
            buf[r, pl.ds(c, 16)] = v


@functools.partial(
    pl.kernel,
    mesh=_MESH,
    out_type=jax.ShapeDtypeStruct((2 * NP,), jnp.float32),
    scratch_types=[
        pltpu.VMEM((40, 128), jnp.int32),
        pltpu.VMEM((1, 128), jnp.float32),
        pltpu.VMEM((1, 640), jnp.float32),
        pltpu.VMEM_SHARED((NP,), jnp.float32),
    ],
)
def _deg_kernel(dst_hbm, deg_out, dbuf, ones_v, zeros_v, deg_sh):
    cid = lax.axis_index("c")
    sid = lax.axis_index("s")
    wid = cid * 16 + sid
    _fill_f32(ones_v, 1, 128, 1.0)
    _fill_f32(zeros_v, 1, 640, 0.0)
    pltpu.sync_copy(zeros_v.at[0], deg_sh.at[pl.ds(sid * 640, 640)])
    plsc.subcore_barrier()

    rowbase = wid * 40
    pltpu.sync_copy(dst_hbm.at[pl.ds(rowbase, 40)], dbuf)

    @pl.loop(0, 40)
    def _(i):
        pltpu.sync_copy(ones_v.at[0], deg_sh.at[dbuf.at[i]], add=True)

    plsc.subcore_barrier()
    pltpu.sync_copy(
        deg_sh.at[pl.ds(sid * 640, 640)],
        deg_out.at[pl.ds(cid * NP + sid * 640, 640)],
    )


def _make_scatter(d):
    @functools.partial(
        pl.kernel,
        mesh=_MESH,
        out_type=jax.ShapeDtypeStruct((2, NP, d), jnp.float32),
        scratch_types=[
            pltpu.VMEM((40, 128), jnp.int32),
            pltpu.VMEM((40, 128), jnp.int32),
            pltpu.VMEM((128, d), jnp.float32),
            pltpu.VMEM((128, d), jnp.float32),
            pltpu.VMEM_SHARED((NP, d), jnp.float32),
            pltpu.SemaphoreType.DMA,
        ],
    )
    def _scatter_kernel(ha, hb, src_hbm, dst_hbm, acc_out, sbuf, dbuf, rows0,
                        rows1, acc_sh, sem):
        cid = lax.axis_index("c")
        sid = lax.axis_index("s")
        _fill_f32(rows0, 128, d, 0.0)
        for k in range(5):
            pltpu.sync_copy(
                rows0, acc_sh.at[pl.ds(sid * 640 + k * 128, 128)])
        plsc.subcore_barrier()

        rowbase = sid * 80

        def pipeline(h):
            for p in range(2):
                pltpu.sync_copy(
                    src_hbm.at[pl.ds(rowbase + p * 40, 40)], sbuf)
                pltpu.sync_copy(
                    dst_hbm.at[pl.ds(rowbase + p * 40, 40)], dbuf)
                pltpu.async_copy(h.at[sbuf.at[0]], rows0, sem)

                @pl.loop(0, 40, step=2)
                def _(i):
                    pltpu.make_async_copy(h.at[sbuf.at[i]], rows0, sem).wait()
                    pltpu.async_copy(h.at[sbuf.at[i + 1]], rows1, sem)
                    pltpu.sync_copy(rows0, acc_sh.at[dbuf.at[i]], add=True)
                    pltpu.make_async_copy(
                        h.at[sbuf.at[i + 1]], rows1, sem).wait()

                    @pl.when(i + 2 < 40)
                    def _():
                        pltpu.async_copy(h.at[sbuf.at[i + 2]], rows0, sem)

                    pltpu.sync_copy(
                        rows1, acc_sh.at[dbuf.at[i + 1]], add=True)

        @pl.when(cid == 0)
        def _():
            pipeline(ha)

        @pl.when(cid == 1)
        def _():
            pipeline(hb)

        plsc.subcore_barrier()
        pltpu.sync_copy(
            acc_sh.at[pl.ds(sid * 640, 640)],
            acc_out.at[cid, pl.ds(sid * 640, 640)],
        )

    return _scatter_kernel


_scatter128 = _make_scatter(128)


@functools.partial(
    pl.kernel,
    mesh=_MESH,
    out_type=jax.ShapeDtypeStruct((2, NP, 128), jnp.float32),
    scratch_types=[
        pltpu.VMEM((40, 128), jnp.int32),
        pltpu.VMEM((40, 128), jnp.int32),
        pltpu.VMEM((128, 128), jnp.float32),
        pltpu.VMEM((128, 128), jnp.float32),
        pltpu.VMEM_SHARED((NP, 128), jnp.float32),
        pltpu.SemaphoreType.DMA,
    ],
)
def _scatter_l2(h, src_hbm, dst_hbm, acc_out, sbuf, dbuf, rows0, rows1,
                acc_sh, sem):
    cid = lax.axis_index("c")
    sid = lax.axis_index("s")
    _fill_f32(rows0, 128, 128, 0.0)
    for k in range(5):
        pltpu.sync_copy(rows0, acc_sh.at[pl.ds(sid * 640 + k * 128, 128)])
    plsc.subcore_barrier()

    rowbase = cid * 640 + sid * 40
    pltpu.sync_copy(src_hbm.at[pl.ds(rowbase, 40)], sbuf)
    pltpu.sync_copy(dst_hbm.at[pl.ds(rowbase, 40)], dbuf)

    pltpu.async_copy(h.at[sbuf.at[0]], rows0, sem)

    @pl.loop(0, 40, step=2)
    def _(i):
        pltpu.make_async_copy(h.at[sbuf.at[i]], rows0, sem).wait()
        pltpu.async_copy(h.at[sbuf.at[i + 1]], rows1, sem)
        pltpu.sync_copy(rows0, acc_sh.at[dbuf.at[i]], add=True)
        pltpu.make_async_copy(h.at[sbuf.at[i + 1]], rows1, sem).wait()

        @pl.when(i + 2 < 40)
        def _():
            pltpu.async_copy(h.at[sbuf.at[i + 2]], rows0, sem)

        pltpu.sync_copy(rows1, acc_sh.at[dbuf.at[i + 1]], add=True)

    plsc.subcore_barrier()
    pltpu.sync_copy(
        acc_sh.at[pl.ds(sid * 640, 640)],
        acc_out.at[cid, pl.ds(sid * 640, 640)],
    )


@functools.partial(
    pl.kernel,
    mesh=_MESH,
    out_type=jax.ShapeDtypeStruct((ER, 128), jnp.float32),
    scratch_types=[
        pltpu.VMEM((40, 128), jnp.int32),
        pltpu.VMEM((40, 128), jnp.int32),
        pltpu.VMEM((128, 128), jnp.float32),
        pltpu.VMEM((128, 128), jnp.float32),
        pltpu.VMEM((128, 128), jnp.float32),
        pltpu.VMEM((128, 128), jnp.float32),
        pltpu.VMEM((16, 17), jnp.float32),
        pltpu.VMEM((40, 128), jnp.float32),
        pltpu.SemaphoreType.DMA,
    ],
    compiler_params=_NO_LAYOUT_CP,
)
def _decoder_kernel(z_hbm, src_hbm, dst_hbm, out_hbm, sbuf, dbuf, zs0, zd0,
                    zs1, zd1, stg, lbuf, sem):
    cid = lax.axis_index("c")
    sid = lax.axis_index("s")
    wid = cid * 16 + sid
    rowbase = wid * 40
    pltpu.sync_copy(src_hbm.at[pl.ds(rowbase, 40)], sbuf)
    pltpu.sync_copy(dst_hbm.at[pl.ds(rowbase, 40)], dbuf)

    lanes = lax.iota(jnp.int32, 16)

    def compute(i, zs, zd):
        for g in range(8):
            @pl.loop(0, 16, unroll=8)
            def _(e):
                r = g * 16 + e
                t0 = zs[r, pl.ds(0, 16)] * zd[r, pl.ds(0, 16)]
                t1 = zs[r, pl.ds(16, 16)] * zd[r, pl.ds(16, 16)]
                t2 = zs[r, pl.ds(32, 16)] * zd[r, pl.ds(32, 16)]
                t3 = zs[r, pl.ds(48, 16)] * zd[r, pl.ds(48, 16)]
                t4 = zs[r, pl.ds(64, 16)] * zd[r, pl.ds(64, 16)]
                t5 = zs[r, pl.ds(80, 16)] * zd[r, pl.ds(80, 16)]
                t6 = zs[r, pl.ds(96, 16)] * zd[r, pl.ds(96, 16)]
                t7 = zs[r, pl.ds(112, 16)] * zd[r, pl.ds(112, 16)]
                t = ((t0 + t1) + (t2 + t3)) + ((t4 + t5) + (t6 + t7))
                plsc.store_scatter(
                    stg, [lanes, jnp.full((16,), e, jnp.int32)], t)

            tot = stg[0, pl.ds(0, 16)]
            for l in range(1, 16):
                tot = tot + stg[l, pl.ds(0, 16)]
            lbuf[i, pl.ds(g * 16, 16)] = 1.0 / (1.0 + jnp.exp(-tot))

    pltpu.async_copy(z_hbm.at[sbuf.at[0]], zs0, sem)
    pltpu.async_copy(z_hbm.at[dbuf.at[0]], zd0, sem)

    @pl.loop(0, 40, step=2)
    def _(i):
        pltpu.make_async_copy(z_hbm.at[sbuf.at[i]], zs0, sem).wait()
        pltpu.make_async_copy(z_hbm.at[dbuf.at[i]], zd0, sem).wait()
        pltpu.async_copy(z_hbm.at[sbuf.at[i + 1]], zs1, sem)
        pltpu.async_copy(z_hbm.at[dbuf.at[i + 1]], zd1, sem)
        compute(i, zs0, zd0)
        pltpu.make_async_copy(z_hbm.at[sbuf.at[i + 1]], zs1, sem).wait()
        pltpu.make_async_copy(z_hbm.at[dbuf.at[i + 1]], zd1, sem).wait()

        @pl.when(i + 2 < 40)
        def _():
            pltpu.async_copy(z_hbm.at[sbuf.at[i + 2]], zs0, sem)
            pltpu.async_copy(z_hbm.at[dbuf.at[i + 2]], zd0, sem)

        compute(i + 1, zs1, zd1)

    pltpu.sync_copy(lbuf, out_hbm.at[pl.ds(rowbase, 40)])


def _mm1_body(x_ref, w_ref, o_ref):
    o_ref[...] = jnp.dot(x_ref[...], w_ref[...],
                         preferred_element_type=jnp.float32)


def _scale1_body(deg_ref, xw_ref, ha_ref, hb_ref, dr_ref):
    deg = deg_ref[0] + deg_ref[1] + 1.0
    dinv = lax.rsqrt(deg)[:, None]
    hs = xw_ref[...] * dinv
    ha_ref[...] = hs[:, :128]
    hb_ref[...] = hs[:, 128:]
    dr_ref[...] = jnp.broadcast_to(dinv, (RB, 128))


def _mid_body(acc_ref, ha_ref, hb_ref, dr_ref, b1_ref, w2_ref, o_ref):
    dinv = dr_ref[...]
    h0 = dinv * (acc_ref[0] + ha_ref[...]) + b1_ref[:, :128]
    h1 = dinv * (acc_ref[1] + hb_ref[...]) + b1_ref[:, 128:]
    out1 = jnp.concatenate([jnp.maximum(h0, 0.0), jnp.maximum(h1, 0.0)],
                           axis=1)
    xw2 = jnp.dot(out1, w2_ref[...], preferred_element_type=jnp.float32)
    o_ref[...] = xw2 * dinv


def _z_body(acc_ref, hs_ref, dr_ref, b2_ref, z_ref):
    z_ref[...] = (dr_ref[...] * (acc_ref[0] + acc_ref[1] + hs_ref[...])
                  + b2_ref[...])


def _rows(shape):
    return pl.BlockSpec((RB,) + shape[1:], lambda i: (i,) + (0,) * (len(shape) - 1))


def kernel(x, edge_index, W1, b1, W2, b2):
    f32 = jnp.float32
    x_p = jnp.pad(x.astype(f32), ((0, NP - N), (0, 0)))
    ep = jnp.concatenate([edge_index, _PAD_EDGES], axis=1)
    src_r = ep[0].reshape(ER, 128)
    dst_r = ep[1].reshape(ER, 128)

    deg2 = _deg_kernel(dst_r).reshape(2, NP)

    xw1 = pl.pallas_call(
        _mm1_body,
        grid=(NB,),
        in_specs=[_rows((NP, D_IN)),
                  pl.BlockSpec((D_IN, D_HID), lambda i: (0, 0))],
        out_specs=_rows((NP, D_HID)),
        out_shape=jax.ShapeDtypeStruct((NP, D_HID), f32),
    )(x_p, W1)

    ha, hb, dr = pl.pallas_call(
        _scale1_body,
        grid=(NB,),
        in_specs=[pl.BlockSpec((2, RB), lambda i: (0, i)),
                  _rows((NP, D_HID))],
        out_specs=[_rows((NP, 128)), _rows((NP, 128)), _rows((NP, 128))],
        out_shape=[jax.ShapeDtypeStruct((NP, 128), f32)] * 3,
    )(deg2, xw1)

    acc1 = _scatter128(ha, hb, src_r, dst_r)

    hs2 = pl.pallas_call(
        _mid_body,
        grid=(NB,),
        in_specs=[pl.BlockSpec((2, RB, 128), lambda i: (0, i, 0)),
                  _rows((NP, 128)), _rows((NP, 128)), _rows((NP, 128)),
                  pl.BlockSpec((1, D_HID), lambda i: (0, 0)),
                  pl.BlockSpec((D_HID, D_OUT), lambda i: (0, 0))],
        out_specs=_rows((NP, 128)),
        out_shape=jax.ShapeDtypeStruct((NP, 128), f32),
    )(acc1, ha, hb, dr, b1[None], W2)

    acc2 = _scatter_l2(hs2, src_r, dst_r)

    z = pl.pallas_call(
        _z_body,
        grid=(NB,),
        in_specs=[pl.BlockSpec((2, RB, 128), lambda i: (0, i, 0)),
                  _rows((NP, 128)), _rows((NP, 128)),
                  pl.BlockSpec((1, D_OUT), lambda i: (0, 0))],
        out_specs=_rows((NP, 128)),
        out_shape=jax.ShapeDtypeStruct((NP, 128), f32),
    )(acc2, hs2, dr, b2[None])

    out2d = _decoder_kernel(z, src_r, dst_r)
    return out2d.reshape(ER * 128)[:E]


# speedup vs baseline: 1.0258x; 1.0258x over previous
import dataclasses
import functools

import jax
import jax.numpy as jnp
import numpy as np
from jax import lax
from jax.experimental import pallas as pl
from jax.experimental.pallas import tpu as pltpu
from jax.experimental.pallas import tpu_sc as plsc

N = 10000
NP = 10240
E = 160000
ER = 1280
D_IN = 256
D_HID = 256
D_OUT = 128
RB = 1024
NB = NP // RB

_MESH = plsc.VectorSubcoreMesh(core_axis_name="c", subcore_axis_name="s")

_PAD_1D = N + np.arange(ER * 128 - E, dtype=np.int32) % (NP - N)
_PAD_EDGES = np.stack([_PAD_1D, _PAD_1D])

_NO_LAYOUT_CP = pltpu.CompilerParams()
if "needs_layout_passes" in pltpu.CompilerParams.__dataclass_fields__:
    _NO_LAYOUT_CP = dataclasses.replace(_NO_LAYOUT_CP,
                                        needs_layout_passes=False)


def _fill_f32(buf, nrows, ncols, val):
    v = jnp.full((16,), val, jnp.float32)

    @pl.loop(0, nrows)
    def _(r):
        @pl.loop(0, ncols, step=16)
        def _(c):
            buf[r, pl.ds(c, 16)] = v


@functools.partial(
    pl.kernel,
    mesh=_MESH,
    out_type=jax.ShapeDtypeStruct((2 * NP,), jnp.float32),
    scratch_types=[
        pltpu.VMEM((40, 128), jnp.int32),
        pltpu.VMEM((1, 128), jnp.float32),
        pltpu.VMEM((1, 640), jnp.float32),
        pltpu.VMEM_SHARED((NP,), jnp.float32),
    ],
)
def _deg_kernel(dst_hbm, deg_out, dbuf, ones_v, zeros_v, deg_sh):
    cid = lax.axis_index("c")
    sid = lax.axis_index("s")
    wid = cid * 16 + sid
    _fill_f32(ones_v, 1, 128, 1.0)
    _fill_f32(zeros_v, 1, 640, 0.0)
    pltpu.sync_copy(zeros_v.at[0], deg_sh.at[pl.ds(sid * 640, 640)])
    plsc.subcore_barrier()

    rowbase = wid * 40
    pltpu.sync_copy(dst_hbm.at[pl.ds(rowbase, 40)], dbuf)

    @pl.loop(0, 40)
    def _(i):
        pltpu.sync_copy(ones_v.at[0], deg_sh.at[dbuf.at[i]], add=True)

    plsc.subcore_barrier()
    pltpu.sync_copy(
        deg_sh.at[pl.ds(sid * 640, 640)],
        deg_out.at[pl.ds(cid * NP + sid * 640, 640)],
    )


def _make_scatter(d):
    @functools.partial(
        pl.kernel,
        mesh=_MESH,
        out_type=jax.ShapeDtypeStruct((2, NP, d), jnp.float32),
        scratch_types=[
            pltpu.VMEM((40, 128), jnp.int32),
            pltpu.VMEM((40, 128), jnp.int32),
            pltpu.VMEM((128, d), jnp.float32),
            pltpu.VMEM((128, d), jnp.float32),
            pltpu.VMEM_SHARED((NP, d), jnp.float32),
            pltpu.SemaphoreType.DMA,
        ],
    )
    def _scatter_kernel(ha, hb, src_hbm, dst_hbm, acc_out, sbuf, dbuf, rows0,
                        rows1, acc_sh, sem):
        cid = lax.axis_index("c")
        sid = lax.axis_index("s")
        _fill_f32(rows0, 128, d, 0.0)
        for k in range(5):
            pltpu.sync_copy(
                rows0, acc_sh.at[pl.ds(sid * 640 + k * 128, 128)])
        plsc.subcore_barrier()

        rowbase = sid * 80

        def pipeline(h):
            for p in range(2):
                pltpu.sync_copy(
                    src_hbm.at[pl.ds(rowbase + p * 40, 40)], sbuf)
                pltpu.sync_copy(
                    dst_hbm.at[pl.ds(rowbase + p * 40, 40)], dbuf)
                pltpu.async_copy(h.at[sbuf.at[0]], rows0, sem)

                @pl.loop(0, 40, step=2)
                def _(i):
                    pltpu.make_async_copy(h.at[sbuf.at[i]], rows0, sem).wait()
                    pltpu.async_copy(h.at[sbuf.at[i + 1]], rows1, sem)
                    pltpu.sync_copy(rows0, acc_sh.at[dbuf.at[i]], add=True)
                    pltpu.make_async_copy(
                        h.at[sbuf.at[i + 1]], rows1, sem).wait()

                    @pl.when(i + 2 < 40)
                    def _():
                        pltpu.async_copy(h.at[sbuf.at[i + 2]], rows0, sem)

                    pltpu.sync_copy(
                        rows1, acc_sh.at[dbuf.at[i + 1]], add=True)

        @pl.when(cid == 0)
        def _():
            pipeline(ha)

        @pl.when(cid == 1)
        def _():
            pipeline(hb)

        plsc.subcore_barrier()
        pltpu.sync_copy(
            acc_sh.at[pl.ds(sid * 640, 640)],
            acc_out.at[cid, pl.ds(sid * 640, 640)],
        )

    return _scatter_kernel


_scatter128 = _make_scatter(128)


@functools.partial(
    pl.kernel,
    mesh=_MESH,
    out_type=jax.ShapeDtypeStruct((2, NP, 128), jnp.float32),
    scratch_types=[
        pltpu.VMEM((40, 128), jnp.int32),
        pltpu.VMEM((40, 128), jnp.int32),
        pltpu.VMEM((128, 128), jnp.float32),
        pltpu.VMEM((128, 128), jnp.float32),
        pltpu.VMEM_SHARED((NP, 128), jnp.float32),
        pltpu.SemaphoreType.DMA,
    ],
)
def _scatter_l2(h, src_hbm, dst_hbm, acc_out, sbuf, dbuf, rows0, rows1,
                acc_sh, sem):
    cid = lax.axis_index("c")
    sid = lax.axis_index("s")
    _fill_f32(rows0, 128, 128, 0.0)
    for k in range(5):
        pltpu.sync_copy(rows0, acc_sh.at[pl.ds(sid * 640 + k * 128, 128)])
    plsc.subcore_barrier()

    rowbase = cid * 640 + sid * 40
    pltpu.sync_copy(src_hbm.at[pl.ds(rowbase, 40)], sbuf)
    pltpu.sync_copy(dst_hbm.at[pl.ds(rowbase, 40)], dbuf)

    pltpu.async_copy(h.at[sbuf.at[0]], rows0, sem)

    @pl.loop(0, 40, step=2)
    def _(i):
        pltpu.make_async_copy(h.at[sbuf.at[i]], rows0, sem).wait()
        pltpu.async_copy(h.at[sbuf.at[i + 1]], rows1, sem)
        pltpu.sync_copy(rows0, acc_sh.at[dbuf.at[i]], add=True)
        pltpu.make_async_copy(h.at[sbuf.at[i + 1]], rows1, sem).wait()

        @pl.when(i + 2 < 40)
        def _():
            pltpu.async_copy(h.at[sbuf.at[i + 2]], rows0, sem)

        pltpu.sync_copy(rows1, acc_sh.at[dbuf.at[i + 1]], add=True)

    plsc.subcore_barrier()
    pltpu.sync_copy(
        acc_sh.at[pl.ds(sid * 640, 640)],
        acc_out.at[cid, pl.ds(sid * 640, 640)],
    )


@functools.partial(
    pl.kernel,
    mesh=_MESH,
    out_type=jax.ShapeDtypeStruct((ER, 128), jnp.float32),
    scratch_types=[
        pltpu.VMEM((40, 128), jnp.int32),
        pltpu.VMEM((40, 128), jnp.int32),
        pltpu.VMEM((128, 128), jnp.float32),
        pltpu.VMEM((128, 128), jnp.float32),
        pltpu.VMEM((128, 128), jnp.float32),
        pltpu.VMEM((128, 128), jnp.float32),
        pltpu.VMEM((16, 17), jnp.float32),
        pltpu.VMEM((40, 128), jnp.float32),
        pltpu.SemaphoreType.DMA,
    ],
    compiler_params=_NO_LAYOUT_CP,
)
def _decoder_kernel(z_hbm, src_hbm, dst_hbm, out_hbm, sbuf, dbuf, zs0, zd0,
                    zs1, zd1, stg, lbuf, sem):
    cid = lax.axis_index("c")
    sid = lax.axis_index("s")
    wid = cid * 16 + sid
    rowbase = wid * 40
    pltpu.sync_copy(src_hbm.at[pl.ds(rowbase, 40)], sbuf)
    pltpu.sync_copy(dst_hbm.at[pl.ds(rowbase, 40)], dbuf)

    lanes = lax.iota(jnp.int32, 16)

    def compute(i, zs, zd):
        for g in range(8):
            @pl.loop(0, 16, unroll=8)
            def _(e):
                r = g * 16 + e
                t0 = zs[r, pl.ds(0, 16)] * zd[r, pl.ds(0, 16)]
                t1 = zs[r, pl.ds(16, 16)] * zd[r, pl.ds(16, 16)]
                t2 = zs[r, pl.ds(32, 16)] * zd[r, pl.ds(32, 16)]
                t3 = zs[r, pl.ds(48, 16)] * zd[r, pl.ds(48, 16)]
                t4 = zs[r, pl.ds(64, 16)] * zd[r, pl.ds(64, 16)]
                t5 = zs[r, pl.ds(80, 16)] * zd[r, pl.ds(80, 16)]
                t6 = zs[r, pl.ds(96, 16)] * zd[r, pl.ds(96, 16)]
                t7 = zs[r, pl.ds(112, 16)] * zd[r, pl.ds(112, 16)]
                t = ((t0 + t1) + (t2 + t3)) + ((t4 + t5) + (t6 + t7))
                plsc.store_scatter(
                    stg, [lanes, jnp.full((16,), e, jnp.int32)], t)

            tot = stg[0, pl.ds(0, 16)]
            for l in range(1, 16):
                tot = tot + stg[l, pl.ds(0, 16)]
            lbuf[i, pl.ds(g * 16, 16)] = 1.0 / (1.0 + jnp.exp(-tot))

    pltpu.async_copy(z_hbm.at[sbuf.at[0]], zs0, sem)
    pltpu.async_copy(z_hbm.at[dbuf.at[0]], zd0, sem)

    @pl.loop(0, 40, step=2)
    def _(i):
        pltpu.make_async_copy(z_hbm.at[sbuf.at[i]], zs0, sem).wait()
        pltpu.make_async_copy(z_hbm.at[dbuf.at[i]], zd0, sem).wait()
        pltpu.async_copy(z_hbm.at[sbuf.at[i + 1]], zs1, sem)
        pltpu.async_copy(z_hbm.at[dbuf.at[i + 1]], zd1, sem)
        compute(i, zs0, zd0)
        pltpu.make_async_copy(z_hbm.at[sbuf.at[i + 1]], zs1, sem).wait()
        pltpu.make_async_copy(z_hbm.at[dbuf.at[i + 1]], zd1, sem).wait()

        @pl.when(i + 2 < 40)
        def _():
            pltpu.async_copy(z_hbm.at[sbuf.at[i + 2]], zs0, sem)
            pltpu.async_copy(z_hbm.at[dbuf.at[i + 2]], zd0, sem)

        compute(i + 1, zs1, zd1)

    pltpu.sync_copy(lbuf, out_hbm.at[pl.ds(rowbase, 40)])


def _enc1_body(deg_ref, x_ref, w_ref, ha_ref, hb_ref, dr_ref):
    deg = deg_ref[0] + deg_ref[1] + 1.0
    dinv = lax.rsqrt(deg)[:, None]
    xw = jnp.dot(x_ref[...], w_ref[...], preferred_element_type=jnp.float32)
    hs = xw * dinv
    ha_ref[...] = hs[:, :128]
    hb_ref[...] = hs[:, 128:]
    dr_ref[...] = jnp.broadcast_to(dinv, (RB, 128))


def _mid_body(acc_ref, ha_ref, hb_ref, dr_ref, b1_ref, w2_ref, o_ref):
    dinv = dr_ref[...]
    h0 = dinv * (acc_ref[0] + ha_ref[...]) + b1_ref[:, :128]
    h1 = dinv * (acc_ref[1] + hb_ref[...]) + b1_ref[:, 128:]
    out1 = jnp.concatenate([jnp.maximum(h0, 0.0), jnp.maximum(h1, 0.0)],
                           axis=1)
    xw2 = jnp.dot(out1, w2_ref[...], preferred_element_type=jnp.float32)
    o_ref[...] = xw2 * dinv


def _z_body(acc_ref, hs_ref, dr_ref, b2_ref, z_ref):
    z_ref[...] = (dr_ref[...] * (acc_ref[0] + acc_ref[1] + hs_ref[...])
                  + b2_ref[...])


def _rows(shape):
    return pl.BlockSpec((RB,) + shape[1:], lambda i: (i,) + (0,) * (len(shape) - 1))


def kernel(x, edge_index, W1, b1, W2, b2):
    f32 = jnp.float32
    x_p = jnp.pad(x.astype(f32), ((0, NP - N), (0, 0)))
    ep = jnp.concatenate([edge_index, _PAD_EDGES], axis=1)
    src_r = ep[0].reshape(ER, 128)
    dst_r = ep[1].reshape(ER, 128)

    deg2 = _deg_kernel(dst_r).reshape(2, NP)

    ha, hb, dr = pl.pallas_call(
        _enc1_body,
        grid=(NB,),
        in_specs=[pl.BlockSpec((2, RB), lambda i: (0, i)),
                  _rows((NP, D_IN)),
                  pl.BlockSpec((D_IN, D_HID), lambda i: (0, 0))],
        out_specs=[_rows((NP, 128)), _rows((NP, 128)), _rows((NP, 128))],
        out_shape=[jax.ShapeDtypeStruct((NP, 128), f32)] * 3,
    )(deg2, x_p, W1)

    acc1 = _scatter128(ha, hb, src_r, dst_r)

    hs2 = pl.pallas_call(
        _mid_body,
        grid=(NB,),
        in_specs=[pl.BlockSpec((2, RB, 128), lambda i: (0, i, 0)),
                  _rows((NP, 128)), _rows((NP, 128)), _rows((NP, 128)),
                  pl.BlockSpec((1, D_HID), lambda i: (0, 0)),
                  pl.BlockSpec((D_HID, D_OUT), lambda i: (0, 0))],
        out_specs=_rows((NP, 128)),
        out_shape=jax.ShapeDtypeStruct((NP, 128), f32),
    )(acc1, ha, hb, dr, b1[None], W2)

    acc2 = _scatter_l2(hs2, src_r, dst_r)

    z = pl.pallas_call(
        _z_body,
        grid=(NB,),
        in_specs=[pl.BlockSpec((2, RB, 128), lambda i: (0, i, 0)),
                  _rows((NP, 128)), _rows((NP, 128)),
                  pl.BlockSpec((1, D_OUT), lambda i: (0, 0))],
        out_specs=_rows((NP, 128)),
        out_shape=jax.ShapeDtypeStruct((NP, 128), f32),
    )(acc2, hs2, dr, b2[None])

    out2d = _decoder_kernel(z, src_r, dst_r)
    return out2d.reshape(ER * 128)[:E]
